# Initial kernel scaffold; baseline (speedup 1.0000x reference)
import jax
import jax.numpy as jnp
from jax.experimental import pallas as pl


def kernel(x, edge_index, batch, batch_size, W1, b1, W2, b2, W3, b3, Wm1, bm1, Wm2, bm2):
    raise NotImplementedError("write your pallas kernel here")


# speedup vs baseline: 15.9631x; 15.9631x over previous
import functools

import jax
import jax.numpy as jnp
from jax import lax
from jax.experimental import pallas as pl
from jax.experimental.pallas import tpu as pltpu
from jax.experimental.pallas import tpu_sc as plsc

N = 10000
NPAD = 10240
E = 320000
D = 128
OUT = 64
G = 64

NC = 2
NS = 16
NW = NC * NS
EPW = E // NW
K = 80
NCHUNK = EPW // K
RPT = NPAD // NS


def _sc_mesh():
    return plsc.VectorSubcoreMesh(core_axis_name="c", subcore_axis_name="s",
                                  num_cores=NC, num_subcores=NS)


@functools.cache
def _make_deg_kernel():
    return pl.kernel(
        _deg_body,
        out_type=jax.ShapeDtypeStruct((NC, NPAD, D), jnp.float32),
        mesh=_sc_mesh(),
        scratch_types=[
            pltpu.VMEM((NCHUNK, K), jnp.int32),
            pltpu.VMEM((K, D), jnp.float32),
            pltpu.VMEM_SHARED((NPAD, D), jnp.float32),
        ],
    )


def _deg_body(dst_hbm, zeros_hbm, ones_hbm, out_hbm, dstw, onesv, degsh):
    c = lax.axis_index("c")
    s = lax.axis_index("s")
    wid = s * NC + c
    pltpu.sync_copy(dst_hbm.at[wid], dstw)
    pltpu.sync_copy(ones_hbm, onesv)
    base = s * RPT
    pltpu.sync_copy(zeros_hbm.at[pl.ds(base, RPT)], degsh.at[pl.ds(base, RPT)])
    plsc.subcore_barrier()

    def body(j, carry):
        pltpu.sync_copy(onesv, degsh.at[dstw.at[j]], add=True)
        return carry

    lax.fori_loop(0, NCHUNK, body, 0)
    plsc.subcore_barrier()
    pltpu.sync_copy(degsh.at[pl.ds(base, RPT)], out_hbm.at[c].at[pl.ds(base, RPT)])


@functools.cache
def _make_agg_kernel():
    return pl.kernel(
        _agg_body,
        out_type=jax.ShapeDtypeStruct((NC, NPAD, D), jnp.float32),
        mesh=_sc_mesh(),
        scratch_types=[
            pltpu.VMEM((NCHUNK, K), jnp.int32),
            pltpu.VMEM((NCHUNK, K), jnp.int32),
            pltpu.VMEM((K, D), jnp.float32),
            pltpu.VMEM_SHARED((NPAD, D), jnp.float32),
            pltpu.SemaphoreType.DMA,
        ],
    )


def _agg_body(y_hbm, zeros_hbm, src_hbm, dst_hbm, out_hbm,
              srcw, dstw, rows, aggsh, gsem):
    c = lax.axis_index("c")
    s = lax.axis_index("s")
    wid = s * NC + c
    pltpu.sync_copy(src_hbm.at[wid], srcw)
    pltpu.sync_copy(dst_hbm.at[wid], dstw)
    base = s * RPT
    @pl.when(c == 0)
    def _():
        pltpu.sync_copy(y_hbm.at[pl.ds(base, RPT)], aggsh.at[pl.ds(base, RPT)])

    @pl.when(c == 1)
    def _():
        pltpu.sync_copy(zeros_hbm.at[pl.ds(base, RPT)], aggsh.at[pl.ds(base, RPT)])

    plsc.subcore_barrier()

    def body(j, carry):
        pltpu.async_copy(y_hbm.at[srcw.at[j]], rows, gsem).wait()
        pltpu.sync_copy(rows, aggsh.at[dstw.at[j]], add=True)
        return carry

    lax.fori_loop(0, NCHUNK, body, 0)
    plsc.subcore_barrier()
    pltpu.sync_copy(aggsh.at[pl.ds(base, RPT)], out_hbm.at[c].at[pl.ds(base, RPT)])


def _gelu(v):
    return 0.5 * v * (1.0 + lax.erf(v * 0.7071067811865476))


def _tc1_body(x_ref, w_ref, deg_ref, y_ref, dinv_ref):
    deg = deg_ref[0, :N, 0] + deg_ref[1, :N, 0] + 1.0
    dinv = lax.rsqrt(deg)
    dinv_ref[...] = dinv[:, None]
    y_ref[:N, :] = (x_ref[...] @ w_ref[...]) * dinv[:, None]
    y_ref[N:, :] = jnp.zeros((NPAD - N, D), jnp.float32)


def _tc1(x, w, deg_parts):
    return pl.pallas_call(
        _tc1_body,
        out_shape=(
            jax.ShapeDtypeStruct((NPAD, D), jnp.float32),
            jax.ShapeDtypeStruct((N, 1), jnp.float32),
        ),
    )(x, w, deg_parts)


def _tc2_body(agg_ref, dinv_ref, b_ref, w_ref, y_ref):
    dinv = dinv_ref[...]
    h = _gelu((agg_ref[0, :N, :] + agg_ref[1, :N, :]) * dinv
              + b_ref[...][None, :])
    y_ref[:N, :] = (h @ w_ref[...]) * dinv
    y_ref[N:, :] = jnp.zeros((NPAD - N, D), jnp.float32)


def _tc2(agg, dinv, b, w):
    return pl.pallas_call(
        _tc2_body,
        out_shape=jax.ShapeDtypeStruct((NPAD, D), jnp.float32),
    )(agg, dinv, b, w)


def _tc3_body(agg_ref, dinv_ref, b_ref, batch_ref, wm1_ref, bm1_ref,
              wm2_ref, bm2_ref, out_ref):
    h = _gelu((agg_ref[0, :N, :] + agg_ref[1, :N, :]) * dinv_ref[...]
              + b_ref[...][None, :])
    gids = lax.broadcasted_iota(jnp.int32, (1, G), 1)
    onehot = (batch_ref[...] == gids).astype(jnp.float32)
    sums = lax.dot_general(onehot, h, (((0,), (0,)), ((), ())))
    counts = jnp.sum(onehot, axis=0)
    pooled = sums / jnp.maximum(counts, 1.0)[:, None]
    hm = jnp.maximum(pooled @ wm1_ref[...] + bm1_ref[...][None, :], 0.0)
    out_ref[...] = hm @ wm2_ref[...] + bm2_ref[...][None, :]


def _tc3(agg, dinv, b, batch2d, wm1, bm1, wm2, bm2):
    return pl.pallas_call(
        _tc3_body,
        out_shape=jax.ShapeDtypeStruct((G, OUT), jnp.float32),
    )(agg, dinv, b, batch2d, wm1, bm1, wm2, bm2)


def kernel(x, edge_index, batch, batch_size, W1, b1, W2, b2, W3, b3,
           Wm1, bm1, Wm2, bm2):
    src = edge_index[0].astype(jnp.int32).reshape(NW, NCHUNK, K)
    dst = edge_index[1].astype(jnp.int32).reshape(NW, NCHUNK, K)
    batch2d = batch.astype(jnp.int32).reshape(N, 1)
    zeros_nd = jnp.zeros((NPAD, D), jnp.float32)
    ones_k = jnp.ones((K, D), jnp.float32)

    deg_parts = _make_deg_kernel()(dst, zeros_nd, ones_k)
    y1, dinv = _tc1(x, W1, deg_parts)
    agg1 = _make_agg_kernel()(y1, zeros_nd, src, dst)
    y2 = _tc2(agg1, dinv, b1, W2)
    agg2 = _make_agg_kernel()(y2, zeros_nd, src, dst)
    y3 = _tc2(agg2, dinv, b2, W3)
    agg3 = _make_agg_kernel()(y3, zeros_nd, src, dst)
    return _tc3(agg3, dinv, b3, batch2d, Wm1, bm1, Wm2, bm2)
